# baseline (speedup 1.0000x reference)
import jax
import jax.numpy as jnp
from jax import lax
from jax.experimental import pallas as pl
from jax.experimental.pallas import tpu as pltpu
from jax.experimental.pallas import tpu_sc as plsc

N = 10000
E = 320000
NFEAT = 128
NHID = 128
NCLASS = 64

NC, NS = 2, 16
NW = NC * NS
CH = 128
NCHUNK = 80
STAGE = 16
TCHUNK = NW * NCHUNK
EPAD = TCHUNK * CH
CF = 1
FSTAGES, SSTAGES = 9, 1
FCHUNKS = NS * FSTAGES * STAGE
NPAD = 10112
SUB_OUT = NPAD // NS
SUB_INIT = 624


_MESH = plsc.VectorSubcoreMesh(core_axis_name="c", subcore_axis_name="s")


def _make_agg(D):
  def body(h, srcs, dsts, pout, src_v, dst_v, r0, r1, acc, sem_a, sem_b):
    c = lax.axis_index("c")
    s = lax.axis_index("s")
    wid = s * NC + c
    pltpu.sync_copy(h.at[pl.ds(s * SUB_INIT, SUB_INIT + 16)],
                    acc.at[pl.ds(s * SUB_INIT, SUB_INIT + 16)])
    plsc.subcore_barrier()

    def wait_gather(buf, sem):
      pltpu.make_async_copy(h.at[pl.ds(0, CH)], buf, sem).wait()

    base = jnp.where(c == CF, s * (FSTAGES * STAGE),
                     FCHUNKS + s * (SSTAGES * STAGE))
    nst = jnp.where(c == CF, FSTAGES, SSTAGES)

    def stage(st, cc):
      off = pl.multiple_of(base + st * STAGE, STAGE)
      pltpu.sync_copy(srcs.at[pl.ds(off, STAGE)], src_v)
      pltpu.sync_copy(dsts.at[pl.ds(off, STAGE)], dst_v)
      pltpu.async_copy(h.at[src_v.at[0]], r0, sem_a)

      def pair(g, c2):
        j0 = 2 * g
        pltpu.async_copy(h.at[src_v.at[j0 + 1]], r1, sem_b)
        wait_gather(r0, sem_a)
        pltpu.sync_copy(r0, acc.at[dst_v.at[j0]], add=True)
        pltpu.async_copy(h.at[src_v.at[j0 + 2]], r0, sem_a)
        wait_gather(r1, sem_b)
        pltpu.sync_copy(r1, acc.at[dst_v.at[j0 + 1]], add=True)
        return c2

      lax.fori_loop(0, STAGE // 2 - 1, pair, 0)
      pltpu.async_copy(h.at[src_v.at[STAGE - 1]], r1, sem_b)
      wait_gather(r0, sem_a)
      pltpu.sync_copy(r0, acc.at[dst_v.at[STAGE - 2]], add=True)
      wait_gather(r1, sem_b)
      pltpu.sync_copy(r1, acc.at[dst_v.at[STAGE - 1]], add=True)
      return cc

    lax.fori_loop(0, nst, stage, 0)
    plsc.subcore_barrier()
    pltpu.sync_copy(acc.at[pl.ds(s * SUB_OUT, SUB_OUT)],
                    pout.at[c].at[pl.ds(s * SUB_OUT, SUB_OUT)])

  return pl.kernel(
      body,
      out_type=jax.ShapeDtypeStruct((NC, NPAD, D), jnp.float32),
      mesh=_MESH,
      scratch_types=(
          pltpu.VMEM((STAGE, CH), jnp.int32),
          pltpu.VMEM((STAGE, CH), jnp.int32),
          pltpu.VMEM((CH, D), jnp.float32),
          pltpu.VMEM((CH, D), jnp.float32),
          pltpu.VMEM_SHARED((NPAD, D), jnp.float32),
          pltpu.SemaphoreType.DMA,
          pltpu.SemaphoreType.DMA,
      ))


def _cnt_kernel():
  def body(zeros_h, ones_h, dsts, cout, dst_v, ones_v, cnt_acc):
    c = lax.axis_index("c")
    s = lax.axis_index("s")
    wid = s * NC + c
    pltpu.sync_copy(zeros_h.at[pl.ds(s * SUB_OUT, SUB_OUT)],
                    cnt_acc.at[pl.ds(s * SUB_OUT, SUB_OUT)])
    pltpu.sync_copy(ones_h, ones_v)
    plsc.subcore_barrier()

    base = wid * NCHUNK
    for st in range(NCHUNK // STAGE):
      off = pl.multiple_of(base + st * STAGE, STAGE)
      pltpu.sync_copy(dsts.at[pl.ds(off, STAGE)], dst_v)

      def step(j, c2):
        pltpu.sync_copy(ones_v, cnt_acc.at[dst_v.at[j]], add=True)
        return c2

      lax.fori_loop(0, STAGE, step, 0)
    plsc.subcore_barrier()
    pltpu.sync_copy(cnt_acc.at[pl.ds(s * SUB_OUT, SUB_OUT)],
                    cout.at[c].at[pl.ds(s * SUB_OUT, SUB_OUT)])

  return pl.kernel(
      body,
      out_type=jax.ShapeDtypeStruct((NC, NPAD, 128), jnp.float32),
      mesh=_MESH,
      scratch_types=(
          pltpu.VMEM((STAGE, CH), jnp.int32),
          pltpu.VMEM((CH, 128), jnp.float32),
          pltpu.VMEM_SHARED((NPAD, 128), jnp.float32),
      ))


_agg1 = _make_agg(NHID)
_agg2 = _make_agg(128)
_cnt = _cnt_kernel()

_BM = 1000


def _mm1(x, w1t, b1):
  def body(x_ref, w_ref, b_ref, o_ref):
    o_ref[...] = jnp.dot(x_ref[...], w_ref[...],
                         preferred_element_type=jnp.float32) + b_ref[...]
  return pl.pallas_call(
      body,
      grid=(N // _BM,),
      in_specs=[pl.BlockSpec((_BM, NFEAT), lambda i: (i, 0)),
                pl.BlockSpec((NFEAT, NHID), lambda i: (0, 0)),
                pl.BlockSpec((1, NHID), lambda i: (0, 0))],
      out_specs=pl.BlockSpec((_BM, NHID), lambda i: (i, 0)),
      out_shape=jax.ShapeDtypeStruct((N, NHID), jnp.float32),
  )(x, w1t, b1.reshape(1, NHID))


def _combine1(p, cnt, h1, w2t, b2):
  def body(p0_ref, p1_ref, c0_ref, c1_ref, h_ref, w_ref, b_ref, o_ref):
    ssum = p0_ref[0] + p1_ref[0] - h_ref[...]
    count = c0_ref[0][:, 0:1] + c1_ref[0][:, 0:1] + 1.0
    a = jnp.maximum(ssum / count, 0.0)
    o_ref[...] = jnp.dot(a, w_ref[...],
                         preferred_element_type=jnp.float32) + b_ref[...]
  return pl.pallas_call(
      body,
      grid=(N // _BM,),
      in_specs=[pl.BlockSpec((1, _BM, NHID), lambda i: (0, i, 0)),
                pl.BlockSpec((1, _BM, NHID), lambda i: (1, i, 0)),
                pl.BlockSpec((1, _BM, 128), lambda i: (0, i, 0)),
                pl.BlockSpec((1, _BM, 128), lambda i: (1, i, 0)),
                pl.BlockSpec((_BM, NHID), lambda i: (i, 0)),
                pl.BlockSpec((NHID, 128), lambda i: (0, 0)),
                pl.BlockSpec((1, 128), lambda i: (0, 0))],
      out_specs=pl.BlockSpec((_BM, 128), lambda i: (i, 0)),
      out_shape=jax.ShapeDtypeStruct((N, 128), jnp.float32),
  )(p, p, cnt, cnt, h1, w2t, b2)


def _combine2(q, cnt, h2):
  def body(q0_ref, q1_ref, c0_ref, c1_ref, h_ref, o_ref):
    ssum = q0_ref[0] + q1_ref[0] - h_ref[...]
    count = c0_ref[0][:, 0:1] + c1_ref[0][:, 0:1] + 1.0
    m = (ssum / count)[:, :NCLASS]
    mx = jnp.max(m, axis=1, keepdims=True)
    z = m - mx
    lse = jnp.log(jnp.sum(jnp.exp(z), axis=1, keepdims=True))
    o_ref[...] = z - lse
  return pl.pallas_call(
      body,
      grid=(N // _BM,),
      in_specs=[pl.BlockSpec((1, _BM, 128), lambda i: (0, i, 0)),
                pl.BlockSpec((1, _BM, 128), lambda i: (1, i, 0)),
                pl.BlockSpec((1, _BM, 128), lambda i: (0, i, 0)),
                pl.BlockSpec((1, _BM, 128), lambda i: (1, i, 0)),
                pl.BlockSpec((_BM, 128), lambda i: (i, 0))],
      out_specs=pl.BlockSpec((_BM, NCLASS), lambda i: (i, 0)),
      out_shape=jax.ShapeDtypeStruct((N, NCLASS), jnp.float32),
  )(q, q, cnt, cnt, h2)


def kernel(x, edge_index, W1, b1, W2, b2):
  ei = edge_index.astype(jnp.int32)
  pad = EPAD - E
  srcp = jnp.concatenate([ei[0], jnp.zeros((pad,), jnp.int32)]).reshape(
      TCHUNK, CH)
  dstp = jnp.concatenate([ei[1], jnp.full((pad,), N, jnp.int32)]).reshape(
      TCHUNK, CH)
  ones128 = jnp.ones((CH, 128), jnp.float32)
  zeros128 = jnp.zeros((NPAD, 128), jnp.float32)

  w2tp = jnp.zeros((NHID, 128), jnp.float32).at[:, :NCLASS].set(W2.T)
  b2p = jnp.zeros((1, 128), jnp.float32).at[:, :NCLASS].set(b2)

  h1 = _mm1(x, W1.T, b1)
  cnt = _cnt(zeros128, ones128, dstp)
  p = _agg1(h1, srcp, dstp)
  h2 = _combine1(p, cnt, h1, w2tp, b2p)
  q = _agg2(h2, srcp, dstp)
  out = _combine2(q, cnt, h2)
  return out


# speedup vs baseline: 1.6867x; 1.6867x over previous
import jax
import jax.numpy as jnp
from jax import lax
from jax.experimental import pallas as pl
from jax.experimental.pallas import tpu as pltpu
from jax.experimental.pallas import tpu_sc as plsc

N = 10000
E = 320000
NFEAT = 128
NHID = 128
NCLASS = 64

NC, NS = 2, 16
NW = NC * NS
CH = 128
NCHUNK = 80
STAGE = 16
TCHUNK = NW * NCHUNK
EPAD = TCHUNK * CH
CF = 1
FSTAGES, SSTAGES = 9, 1
FCHUNKS = NS * FSTAGES * STAGE
NPAD = 10112
SUB_OUT = NPAD // NS
SUB_INIT = 624


_MESH = plsc.VectorSubcoreMesh(core_axis_name="c", subcore_axis_name="s")


def _make_agg(D):
  def body(h, srcs, dsts, pout, src_v, dst_v, r0, r1, acc, sem_a, sem_b):
    c = lax.axis_index("c")
    s = lax.axis_index("s")
    wid = s * NC + c
    pltpu.sync_copy(h.at[pl.ds(s * SUB_INIT, SUB_INIT + 16)],
                    acc.at[pl.ds(s * SUB_INIT, SUB_INIT + 16)])
    plsc.subcore_barrier()

    def wait_gather(buf, sem):
      pltpu.make_async_copy(h.at[pl.ds(0, CH)], buf, sem).wait()

    base = jnp.where(c == CF, s * (FSTAGES * STAGE),
                     FCHUNKS + s * (SSTAGES * STAGE))
    nst = jnp.where(c == CF, FSTAGES, SSTAGES)

    def stage(st, cc):
      off = pl.multiple_of(base + st * STAGE, STAGE)
      pltpu.sync_copy(srcs.at[pl.ds(off, STAGE)], src_v)
      pltpu.sync_copy(dsts.at[pl.ds(off, STAGE)], dst_v)
      pltpu.async_copy(h.at[src_v.at[0]], r0, sem_a)

      def pair(g, c2):
        j0 = 2 * g
        pltpu.async_copy(h.at[src_v.at[j0 + 1]], r1, sem_b)
        wait_gather(r0, sem_a)
        pltpu.sync_copy(r0, acc.at[dst_v.at[j0]], add=True)
        pltpu.async_copy(h.at[src_v.at[j0 + 2]], r0, sem_a)
        wait_gather(r1, sem_b)
        pltpu.sync_copy(r1, acc.at[dst_v.at[j0 + 1]], add=True)
        return c2

      lax.fori_loop(0, STAGE // 2 - 1, pair, 0)
      pltpu.async_copy(h.at[src_v.at[STAGE - 1]], r1, sem_b)
      wait_gather(r0, sem_a)
      pltpu.sync_copy(r0, acc.at[dst_v.at[STAGE - 2]], add=True)
      wait_gather(r1, sem_b)
      pltpu.sync_copy(r1, acc.at[dst_v.at[STAGE - 1]], add=True)
      return cc

    lax.fori_loop(0, nst, stage, 0)
    plsc.subcore_barrier()
    pltpu.sync_copy(acc.at[pl.ds(s * SUB_OUT, SUB_OUT)],
                    pout.at[c].at[pl.ds(s * SUB_OUT, SUB_OUT)])

  return pl.kernel(
      body,
      out_type=jax.ShapeDtypeStruct((NC, NPAD, D), jnp.float32),
      mesh=_MESH,
      scratch_types=(
          pltpu.VMEM((STAGE, CH), jnp.int32),
          pltpu.VMEM((STAGE, CH), jnp.int32),
          pltpu.VMEM((CH, D), jnp.float32),
          pltpu.VMEM((CH, D), jnp.float32),
          pltpu.VMEM_SHARED((NPAD, D), jnp.float32),
          pltpu.SemaphoreType.DMA,
          pltpu.SemaphoreType.DMA,
      ))


def _cnt_kernel():
  def body(zeros_h, ones_h, dsts, cout, dst_v, ones_v, cnt_acc):
    c = lax.axis_index("c")
    s = lax.axis_index("s")
    wid = s * NC + c
    pltpu.sync_copy(zeros_h.at[pl.ds(s * SUB_OUT, SUB_OUT)],
                    cnt_acc.at[pl.ds(s * SUB_OUT, SUB_OUT)])
    pltpu.sync_copy(ones_h, ones_v)
    plsc.subcore_barrier()

    base = wid * NCHUNK
    for st in range(NCHUNK // STAGE):
      off = pl.multiple_of(base + st * STAGE, STAGE)
      pltpu.sync_copy(dsts.at[pl.ds(off, STAGE)], dst_v)

      def step(j, c2):
        pltpu.sync_copy(ones_v, cnt_acc.at[dst_v.at[j]], add=True)
        return c2

      lax.fori_loop(0, STAGE, step, 0)
    plsc.subcore_barrier()
    pltpu.sync_copy(cnt_acc.at[pl.ds(s * SUB_OUT, SUB_OUT)],
                    cout.at[c].at[pl.ds(s * SUB_OUT, SUB_OUT)])

  return pl.kernel(
      body,
      out_type=jax.ShapeDtypeStruct((NC, NPAD, 128), jnp.float32),
      mesh=_MESH,
      scratch_types=(
          pltpu.VMEM((STAGE, CH), jnp.int32),
          pltpu.VMEM((CH, 128), jnp.float32),
          pltpu.VMEM_SHARED((NPAD, 128), jnp.float32),
      ))


_agg1 = _make_agg(NHID)
_agg2 = _make_agg(128)
_cnt = _cnt_kernel()

_BM = 1000


def _mm1(x, w1t, b1):
  def body(x_ref, w_ref, b_ref, o_ref):
    o_ref[...] = jnp.dot(x_ref[...], w_ref[...],
                         preferred_element_type=jnp.float32) + b_ref[...]
  return pl.pallas_call(
      body,
      grid=(N // _BM,),
      in_specs=[pl.BlockSpec((_BM, NFEAT), lambda i: (i, 0)),
                pl.BlockSpec((NFEAT, NHID), lambda i: (0, 0)),
                pl.BlockSpec((1, NHID), lambda i: (0, 0))],
      out_specs=pl.BlockSpec((_BM, NHID), lambda i: (i, 0)),
      out_shape=jax.ShapeDtypeStruct((N, NHID), jnp.float32),
  )(x, w1t, b1.reshape(1, NHID))


def _combine1(p, cnt, h1, w2t, b2):
  def body(p0_ref, p1_ref, c0_ref, c1_ref, h_ref, w_ref, b_ref, o_ref):
    ssum = p0_ref[0] + p1_ref[0] - h_ref[...]
    count = c0_ref[0][:, 0:1] + c1_ref[0][:, 0:1] + 1.0
    a = jnp.maximum(ssum / count, 0.0)
    o_ref[...] = jnp.dot(a, w_ref[...],
                         preferred_element_type=jnp.float32) + b_ref[...]
  return pl.pallas_call(
      body,
      grid=(N // _BM,),
      in_specs=[pl.BlockSpec((1, _BM, NHID), lambda i: (0, i, 0)),
                pl.BlockSpec((1, _BM, NHID), lambda i: (1, i, 0)),
                pl.BlockSpec((1, _BM, 128), lambda i: (0, i, 0)),
                pl.BlockSpec((1, _BM, 128), lambda i: (1, i, 0)),
                pl.BlockSpec((_BM, NHID), lambda i: (i, 0)),
                pl.BlockSpec((NHID, 128), lambda i: (0, 0)),
                pl.BlockSpec((1, 128), lambda i: (0, 0))],
      out_specs=pl.BlockSpec((_BM, 128), lambda i: (i, 0)),
      out_shape=jax.ShapeDtypeStruct((N, 128), jnp.float32),
  )(p, p, cnt, cnt, h1, w2t, b2)


def _combine2(q, cnt, h2):
  def body(q0_ref, q1_ref, c0_ref, c1_ref, h_ref, o_ref):
    ssum = q0_ref[0] + q1_ref[0] - h_ref[...]
    count = c0_ref[0][:, 0:1] + c1_ref[0][:, 0:1] + 1.0
    m = (ssum / count)[:, :NCLASS]
    mx = jnp.max(m, axis=1, keepdims=True)
    z = m - mx
    lse = jnp.log(jnp.sum(jnp.exp(z), axis=1, keepdims=True))
    o_ref[...] = z - lse
  return pl.pallas_call(
      body,
      grid=(N // _BM,),
      in_specs=[pl.BlockSpec((1, _BM, 128), lambda i: (0, i, 0)),
                pl.BlockSpec((1, _BM, 128), lambda i: (1, i, 0)),
                pl.BlockSpec((1, _BM, 128), lambda i: (0, i, 0)),
                pl.BlockSpec((1, _BM, 128), lambda i: (1, i, 0)),
                pl.BlockSpec((_BM, 128), lambda i: (i, 0))],
      out_specs=pl.BlockSpec((_BM, NCLASS), lambda i: (i, 0)),
      out_shape=jax.ShapeDtypeStruct((N, NCLASS), jnp.float32),
  )(q, q, cnt, cnt, h2)


def kernel(x, edge_index, W1, b1, W2, b2):
  ei = edge_index.astype(jnp.int32)
  pad = EPAD - E
  srcp = jnp.concatenate(
      [ei[0], (jnp.arange(pad, dtype=jnp.int32) * 7) % N]).reshape(TCHUNK, CH)
  dstp = jnp.concatenate([ei[1], jnp.full((pad,), N, jnp.int32)]).reshape(
      TCHUNK, CH)
  ones128 = jnp.ones((CH, 128), jnp.float32)
  zeros128 = jnp.zeros((NPAD, 128), jnp.float32)

  w2tp = jnp.zeros((NHID, 128), jnp.float32).at[:, :NCLASS].set(W2.T)
  b2p = jnp.zeros((1, 128), jnp.float32).at[:, :NCLASS].set(b2)

  h1 = _mm1(x, W1.T, b1)
  cnt = _cnt(zeros128, ones128, dstp)
  p = _agg1(h1, srcp, dstp)
  h2 = _combine1(p, cnt, h1, w2tp, b2p)
  q = _agg2(h2, srcp, dstp)
  out = _combine2(q, cnt, h2)
  return out


# speedup vs baseline: 2.3673x; 1.4035x over previous
import jax
import jax.numpy as jnp
from jax import lax
from jax.experimental import pallas as pl
from jax.experimental.pallas import tpu as pltpu
from jax.experimental.pallas import tpu_sc as plsc

N = 10000
E = 320000
NFEAT = 128
NHID = 128
NCLASS = 64

NC, NS = 2, 16
NW = NC * NS
CH = 128
NCHUNK = 80
STAGE = 16
TCHUNK = NW * NCHUNK
EPAD = TCHUNK * CH
CF = 1
FSTAGES, SSTAGES = 5, 5
FCHUNKS = NS * FSTAGES * STAGE
NPAD = 10112
SUB_OUT = NPAD // NS
SUB_INIT = 624


_MESH = plsc.VectorSubcoreMesh(core_axis_name="c", subcore_axis_name="s")


def _make_agg(D):
  def body(h, srcs, dsts, pout, src_v, dst_v, r0, r1, acc, sem_a, sem_b):
    c = lax.axis_index("c")
    s = lax.axis_index("s")
    wid = s * NC + c
    pltpu.sync_copy(h.at[pl.ds(s * SUB_INIT, SUB_INIT + 16)],
                    acc.at[pl.ds(s * SUB_INIT, SUB_INIT + 16)])
    plsc.subcore_barrier()

    def wait_gather(buf, sem):
      pltpu.make_async_copy(h.at[pl.ds(0, CH)], buf, sem).wait()

    base = jnp.where(c == CF, s * (FSTAGES * STAGE),
                     FCHUNKS + s * (SSTAGES * STAGE))
    nst = jnp.where(c == CF, FSTAGES, SSTAGES)

    def stage(st, cc):
      off = pl.multiple_of(base + st * STAGE, STAGE)
      pltpu.sync_copy(srcs.at[pl.ds(off, STAGE)], src_v)
      pltpu.sync_copy(dsts.at[pl.ds(off, STAGE)], dst_v)
      pltpu.async_copy(h.at[src_v.at[0]], r0, sem_a)

      def pair(g, c2):
        j0 = 2 * g
        pltpu.async_copy(h.at[src_v.at[j0 + 1]], r1, sem_b)
        wait_gather(r0, sem_a)
        pltpu.sync_copy(r0, acc.at[dst_v.at[j0]], add=True)
        pltpu.async_copy(h.at[src_v.at[j0 + 2]], r0, sem_a)
        wait_gather(r1, sem_b)
        pltpu.sync_copy(r1, acc.at[dst_v.at[j0 + 1]], add=True)
        return c2

      lax.fori_loop(0, STAGE // 2 - 1, pair, 0)
      pltpu.async_copy(h.at[src_v.at[STAGE - 1]], r1, sem_b)
      wait_gather(r0, sem_a)
      pltpu.sync_copy(r0, acc.at[dst_v.at[STAGE - 2]], add=True)
      wait_gather(r1, sem_b)
      pltpu.sync_copy(r1, acc.at[dst_v.at[STAGE - 1]], add=True)
      return cc

    lax.fori_loop(0, nst, stage, 0)
    plsc.subcore_barrier()
    pltpu.sync_copy(acc.at[pl.ds(s * SUB_OUT, SUB_OUT)],
                    pout.at[c].at[pl.ds(s * SUB_OUT, SUB_OUT)])

  return pl.kernel(
      body,
      out_type=jax.ShapeDtypeStruct((NC, NPAD, D), jnp.float32),
      mesh=_MESH,
      scratch_types=(
          pltpu.VMEM((STAGE, CH), jnp.int32),
          pltpu.VMEM((STAGE, CH), jnp.int32),
          pltpu.VMEM((CH, D), jnp.float32),
          pltpu.VMEM((CH, D), jnp.float32),
          pltpu.VMEM_SHARED((NPAD, D), jnp.float32),
          pltpu.SemaphoreType.DMA,
          pltpu.SemaphoreType.DMA,
      ))


def _cnt_kernel():
  def body(zeros_h, ones_h, dsts, cout, dst_v, ones_v, cnt_acc):
    c = lax.axis_index("c")
    s = lax.axis_index("s")
    wid = s * NC + c
    pltpu.sync_copy(zeros_h.at[pl.ds(s * SUB_OUT, SUB_OUT)],
                    cnt_acc.at[pl.ds(s * SUB_OUT, SUB_OUT)])
    pltpu.sync_copy(ones_h, ones_v)
    plsc.subcore_barrier()

    base = wid * NCHUNK
    for st in range(NCHUNK // STAGE):
      off = pl.multiple_of(base + st * STAGE, STAGE)
      pltpu.sync_copy(dsts.at[pl.ds(off, STAGE)], dst_v)

      def step(j, c2):
        pltpu.sync_copy(ones_v, cnt_acc.at[dst_v.at[j]], add=True)
        return c2

      lax.fori_loop(0, STAGE, step, 0)
    plsc.subcore_barrier()
    pltpu.sync_copy(cnt_acc.at[pl.ds(s * SUB_OUT, SUB_OUT)],
                    cout.at[c].at[pl.ds(s * SUB_OUT, SUB_OUT)])

  return pl.kernel(
      body,
      out_type=jax.ShapeDtypeStruct((NC, NPAD, 128), jnp.float32),
      mesh=_MESH,
      scratch_types=(
          pltpu.VMEM((STAGE, CH), jnp.int32),
          pltpu.VMEM((CH, 128), jnp.float32),
          pltpu.VMEM_SHARED((NPAD, 128), jnp.float32),
      ))


_agg1 = _make_agg(NHID)
_agg2 = _make_agg(128)
_cnt = _cnt_kernel()

_BM = 1000


def _mm1(x, w1t, b1):
  def body(x_ref, w_ref, b_ref, o_ref):
    o_ref[...] = jnp.dot(x_ref[...], w_ref[...],
                         preferred_element_type=jnp.float32) + b_ref[...]
  return pl.pallas_call(
      body,
      grid=(N // _BM,),
      in_specs=[pl.BlockSpec((_BM, NFEAT), lambda i: (i, 0)),
                pl.BlockSpec((NFEAT, NHID), lambda i: (0, 0)),
                pl.BlockSpec((1, NHID), lambda i: (0, 0))],
      out_specs=pl.BlockSpec((_BM, NHID), lambda i: (i, 0)),
      out_shape=jax.ShapeDtypeStruct((N, NHID), jnp.float32),
  )(x, w1t, b1.reshape(1, NHID))


def _combine1(p, cnt, h1, w2t, b2):
  def body(p0_ref, p1_ref, c0_ref, c1_ref, h_ref, w_ref, b_ref, o_ref):
    ssum = p0_ref[0] + p1_ref[0] - h_ref[...]
    count = c0_ref[0][:, 0:1] + c1_ref[0][:, 0:1] + 1.0
    a = jnp.maximum(ssum / count, 0.0)
    o_ref[...] = jnp.dot(a, w_ref[...],
                         preferred_element_type=jnp.float32) + b_ref[...]
  return pl.pallas_call(
      body,
      grid=(N // _BM,),
      in_specs=[pl.BlockSpec((1, _BM, NHID), lambda i: (0, i, 0)),
                pl.BlockSpec((1, _BM, NHID), lambda i: (1, i, 0)),
                pl.BlockSpec((1, _BM, 128), lambda i: (0, i, 0)),
                pl.BlockSpec((1, _BM, 128), lambda i: (1, i, 0)),
                pl.BlockSpec((_BM, NHID), lambda i: (i, 0)),
                pl.BlockSpec((NHID, 128), lambda i: (0, 0)),
                pl.BlockSpec((1, 128), lambda i: (0, 0))],
      out_specs=pl.BlockSpec((_BM, 128), lambda i: (i, 0)),
      out_shape=jax.ShapeDtypeStruct((N, 128), jnp.float32),
  )(p, p, cnt, cnt, h1, w2t, b2)


def _combine2(q, cnt, h2):
  def body(q0_ref, q1_ref, c0_ref, c1_ref, h_ref, o_ref):
    ssum = q0_ref[0] + q1_ref[0] - h_ref[...]
    count = c0_ref[0][:, 0:1] + c1_ref[0][:, 0:1] + 1.0
    m = (ssum / count)[:, :NCLASS]
    mx = jnp.max(m, axis=1, keepdims=True)
    z = m - mx
    lse = jnp.log(jnp.sum(jnp.exp(z), axis=1, keepdims=True))
    o_ref[...] = z - lse
  return pl.pallas_call(
      body,
      grid=(N // _BM,),
      in_specs=[pl.BlockSpec((1, _BM, 128), lambda i: (0, i, 0)),
                pl.BlockSpec((1, _BM, 128), lambda i: (1, i, 0)),
                pl.BlockSpec((1, _BM, 128), lambda i: (0, i, 0)),
                pl.BlockSpec((1, _BM, 128), lambda i: (1, i, 0)),
                pl.BlockSpec((_BM, 128), lambda i: (i, 0))],
      out_specs=pl.BlockSpec((_BM, NCLASS), lambda i: (i, 0)),
      out_shape=jax.ShapeDtypeStruct((N, NCLASS), jnp.float32),
  )(q, q, cnt, cnt, h2)


def kernel(x, edge_index, W1, b1, W2, b2):
  ei = edge_index.astype(jnp.int32)
  pad = EPAD - E
  srcp = jnp.concatenate(
      [ei[0], (jnp.arange(pad, dtype=jnp.int32) * 7) % N]).reshape(TCHUNK, CH)
  dstp = jnp.concatenate([ei[1], jnp.full((pad,), N, jnp.int32)]).reshape(
      TCHUNK, CH)
  ones128 = jnp.ones((CH, 128), jnp.float32)
  zeros128 = jnp.zeros((NPAD, 128), jnp.float32)

  w2tp = jnp.zeros((NHID, 128), jnp.float32).at[:, :NCLASS].set(W2.T)
  b2p = jnp.zeros((1, 128), jnp.float32).at[:, :NCLASS].set(b2)

  h1 = _mm1(x, W1.T, b1)
  cnt = _cnt(zeros128, ones128, dstp)
  p = _agg1(h1, srcp, dstp)
  h2 = _combine1(p, cnt, h1, w2tp, b2p)
  q = _agg2(h2, srcp, dstp)
  out = _combine2(q, cnt, h2)
  return out
